# baseline (speedup 1.0000x reference)
import jax
import jax.numpy as jnp
from jax import lax
from jax.experimental import pallas as pl
from jax.experimental.pallas import tpu as pltpu
from jax.experimental.pallas import tpu_sc as plsc

DIM = 128
NCTX = 5
NLANE = 16
NC = 2
NS = 16
NW = NC * NS
C = 64
NCHUNK = 8

_RB, _RC, _RD, _RE = 256, 400, 488, 544
_RED_WORDS = 576


def _tree_reduce(red_v, rb, out_v, out_off):
    for k in range(NLANE):
        red_v[pl.ds(rb + _RB + 8 * k, NLANE)] = (
            red_v[pl.ds(rb + 16 * k, NLANE)]
            + red_v[pl.ds(rb + 16 * k + 8, NLANE)])
    for k in range(NLANE):
        red_v[pl.ds(rb + _RC + 4 * k, NLANE)] = (
            red_v[pl.ds(rb + _RB + 8 * k, NLANE)]
            + red_v[pl.ds(rb + _RB + 8 * k + 4, NLANE)])
    for k in range(NLANE):
        red_v[pl.ds(rb + _RD + 2 * k, NLANE)] = (
            red_v[pl.ds(rb + _RC + 4 * k, NLANE)]
            + red_v[pl.ds(rb + _RC + 4 * k + 2, NLANE)])
    for k in range(NLANE):
        red_v[pl.ds(rb + _RE + k, NLANE)] = (
            red_v[pl.ds(rb + _RD + 2 * k, NLANE)]
            + red_v[pl.ds(rb + _RD + 2 * k + 1, NLANE)])
    out_v[pl.ds(out_off, NLANE)] = red_v[pl.ds(rb + _RE, NLANE)]


def _sc_body(tgt_idx_hbm, ctx_idx_hbm, tgt_tab_hbm, ctx_tab_hbm, out_hbm,
             tgtidx_v, ctxidx_v, tgt_rows_v, ctx_rows_v, out_v, red_v,
             sem0, sem1):
    bpw = C * NCHUNK
    wid = lax.axis_index("s") * NC + lax.axis_index("c")

    pltpu.sync_copy(tgt_idx_hbm.at[pl.ds(wid * NCHUNK, NCHUNK)], tgtidx_v)
    pltpu.sync_copy(ctx_idx_hbm.at[pl.ds(wid * NCHUNK * NCTX, NCHUNK * NCTX)],
                    ctxidx_v)

    def issue(s, poff, sem):
        pltpu.async_copy(tgt_tab_hbm.at[tgtidx_v.at[s]],
                         tgt_rows_v.at[pl.ds(poff, C)], sem)
        for j in range(NCTX):
            pltpu.async_copy(
                ctx_tab_hbm.at[ctxidx_v.at[NCTX * s + j]],
                ctx_rows_v.at[pl.ds(poff * NCTX + j * C, C)], sem)

    def drain(s, poff, sem):
        pltpu.make_async_copy(tgt_tab_hbm.at[tgtidx_v.at[s]],
                              tgt_rows_v.at[pl.ds(poff, C)], sem).wait()
        for j in range(NCTX):
            pltpu.make_async_copy(
                ctx_tab_hbm.at[ctxidx_v.at[NCTX * s + j]],
                ctx_rows_v.at[pl.ds(poff * NCTX + j * C, C)], sem).wait()

    issue(0, 0, sem0)

    def outer(s, carry):
        p = s & 1
        poff = p * C
        poff5 = p * C * NCTX

        @pl.when(p == 0)
        def _():
            drain(s, poff, sem0)

            @pl.when(s < NCHUNK - 1)
            def _():
                issue(s + 1, C, sem1)

        @pl.when(p == 1)
        def _():
            drain(s, poff, sem1)

            @pl.when(s < NCHUNK - 1)
            def _():
                issue(s + 1, 0, sem0)

        @plsc.parallel_loop(0, C // NLANE, unroll=2)
        def gb_body(gb):
            rb = gb * _RED_WORDS
            for i_off in range(NLANE):
                i = gb * NLANE + i_off
                we = [tgt_rows_v[poff + i, pl.ds(k * NLANE, NLANE)]
                      for k in range(DIM // NLANE)]
                for c in range(NCTX):
                    td = i_off * NCTX + c
                    q = i * NCTX + c
                    acc = we[0] * ctx_rows_v[poff5 + q, pl.ds(0, NLANE)]
                    for k in range(1, DIM // NLANE):
                        acc = acc + we[k] * ctx_rows_v[poff5 + q,
                                                       pl.ds(k * NLANE, NLANE)]
                    red_v[pl.ds(rb + 16 * (td % NLANE), NLANE)] = acc
                    if td % NLANE == NLANE - 1:
                        _tree_reduce(red_v, rb, out_v,
                                     s * C * NCTX + gb * NLANE * NCTX
                                     + (td // NLANE) * NLANE)
        return carry

    lax.fori_loop(0, NCHUNK, outer, 0)
    pltpu.sync_copy(out_v, out_hbm.at[pl.ds(wid * bpw * NCTX, bpw * NCTX)])


def kernel(target, context, target_table, context_table):
    batch = target.shape[0]
    tgt_idx = target.reshape(batch // C, C)
    ctx_idx = context.reshape(batch * NCTX // C, C)
    mesh = plsc.VectorSubcoreMesh(core_axis_name="c", subcore_axis_name="s")
    out_flat = pl.kernel(
        _sc_body,
        out_type=jax.ShapeDtypeStruct((batch * NCTX,), jnp.float32),
        mesh=mesh,
        scratch_types=[
            pltpu.VMEM((NCHUNK, C), jnp.int32),
            pltpu.VMEM((NCHUNK * NCTX, C), jnp.int32),
            pltpu.VMEM((2 * C, DIM), jnp.float32),
            pltpu.VMEM((2 * C * NCTX, DIM), jnp.float32),
            pltpu.VMEM((C * NCHUNK * NCTX,), jnp.float32),
            pltpu.VMEM(((C // NLANE) * _RED_WORDS,), jnp.float32),
            pltpu.SemaphoreType.DMA,
            pltpu.SemaphoreType.DMA,
        ],
    )(tgt_idx, ctx_idx, target_table, context_table)
    return out_flat.reshape(batch, NCTX)


# speedup vs baseline: 1.0737x; 1.0737x over previous
import jax
import jax.numpy as jnp
from jax import lax
from jax.experimental import pallas as pl
from jax.experimental.pallas import tpu as pltpu
from jax.experimental.pallas import tpu_sc as plsc

DIM = 128
NCTX = 5
NLANE = 16
NC = 2
NS = 16
NW = NC * NS
C = 64
NCHUNK = 8

_RB, _RC, _RD, _RE = 256, 400, 488, 544
_RED_WORDS = 576


def _tree_reduce(red_v, out_v, out_off):
    for k in range(NLANE):
        red_v[pl.ds(_RB + 8 * k, NLANE)] = (
            red_v[pl.ds(16 * k, NLANE)] + red_v[pl.ds(16 * k + 8, NLANE)])
    for k in range(NLANE):
        red_v[pl.ds(_RC + 4 * k, NLANE)] = (
            red_v[pl.ds(_RB + 8 * k, NLANE)]
            + red_v[pl.ds(_RB + 8 * k + 4, NLANE)])
    for k in range(NLANE):
        red_v[pl.ds(_RD + 2 * k, NLANE)] = (
            red_v[pl.ds(_RC + 4 * k, NLANE)]
            + red_v[pl.ds(_RC + 4 * k + 2, NLANE)])
    for k in range(NLANE):
        red_v[pl.ds(_RE + k, NLANE)] = (
            red_v[pl.ds(_RD + 2 * k, NLANE)]
            + red_v[pl.ds(_RD + 2 * k + 1, NLANE)])
    out_v[pl.ds(out_off, NLANE)] = red_v[pl.ds(_RE, NLANE)]


def _sc_body(tgt_idx_hbm, ctx_idx_hbm, tgt_tab_hbm, ctx_tab_hbm, out_hbm,
             tgtidx_v, ctxidx_v, tgt_rows0, tgt_rows1, ctx_rows0, ctx_rows1,
             out_v, red_v, sem0, sem1):
    bpw = C * NCHUNK
    wid = lax.axis_index("s") * NC + lax.axis_index("c")

    tgt_rows = (tgt_rows0, tgt_rows1)
    ctx_rows = (ctx_rows0, ctx_rows1)
    sems = (sem0, sem1)

    pltpu.sync_copy(tgt_idx_hbm.at[pl.ds(wid * NCHUNK, NCHUNK)], tgtidx_v)
    pltpu.sync_copy(ctx_idx_hbm.at[pl.ds(wid * NCHUNK * NCTX, NCHUNK * NCTX)],
                    ctxidx_v)

    def issue(s, p):
        pltpu.async_copy(tgt_tab_hbm.at[tgtidx_v.at[s]], tgt_rows[p], sems[p])
        for j in range(NCTX):
            pltpu.async_copy(ctx_tab_hbm.at[ctxidx_v.at[NCTX * s + j]],
                             ctx_rows[p].at[pl.ds(j * C, C)], sems[p])

    def drain(s, p):
        pltpu.make_async_copy(tgt_tab_hbm.at[tgtidx_v.at[s]], tgt_rows[p],
                              sems[p]).wait()
        for j in range(NCTX):
            pltpu.make_async_copy(ctx_tab_hbm.at[ctxidx_v.at[NCTX * s + j]],
                                  ctx_rows[p].at[pl.ds(j * C, C)],
                                  sems[p]).wait()

    def compute(s, p):
        trows, crows = tgt_rows[p], ctx_rows[p]

        def gb_body(gb, carry):
            for i_off in range(NLANE):
                i = gb * NLANE + i_off
                we = [trows[i, pl.ds(k * NLANE, NLANE)]
                      for k in range(DIM // NLANE)]
                for c in range(NCTX):
                    td = i_off * NCTX + c
                    q = i * NCTX + c
                    acc = we[0] * crows[q, pl.ds(0, NLANE)]
                    for k in range(1, DIM // NLANE):
                        acc = acc + we[k] * crows[q, pl.ds(k * NLANE, NLANE)]
                    red_v[pl.ds(16 * (td % NLANE), NLANE)] = acc
                    if td % NLANE == NLANE - 1:
                        _tree_reduce(red_v, out_v,
                                     s * C * NCTX + gb * NLANE * NCTX
                                     + (td // NLANE) * NLANE)
            return carry

        lax.fori_loop(0, C // NLANE, gb_body, 0)

    issue(0, 0)

    def outer(s2, carry):
        for p in (0, 1):
            s = 2 * s2 + p
            drain(s, p)
            if p == 0:
                issue(s + 1, 1)
            else:
                @pl.when(s2 < NCHUNK // 2 - 1)
                def _():
                    issue(s + 1, 0)
            compute(s, p)
        return carry

    lax.fori_loop(0, NCHUNK // 2, outer, 0)
    pltpu.sync_copy(out_v, out_hbm.at[pl.ds(wid * bpw * NCTX, bpw * NCTX)])


def kernel(target, context, target_table, context_table):
    batch = target.shape[0]
    tgt_idx = target.reshape(batch // C, C)
    ctx_idx = context.reshape(batch * NCTX // C, C)
    mesh = plsc.VectorSubcoreMesh(core_axis_name="c", subcore_axis_name="s")
    out_flat = pl.kernel(
        _sc_body,
        out_type=jax.ShapeDtypeStruct((batch * NCTX,), jnp.float32),
        mesh=mesh,
        scratch_types=[
            pltpu.VMEM((NCHUNK, C), jnp.int32),
            pltpu.VMEM((NCHUNK * NCTX, C), jnp.int32),
            pltpu.VMEM((C, DIM), jnp.float32),
            pltpu.VMEM((C, DIM), jnp.float32),
            pltpu.VMEM((C * NCTX, DIM), jnp.float32),
            pltpu.VMEM((C * NCTX, DIM), jnp.float32),
            pltpu.VMEM((C * NCHUNK * NCTX,), jnp.float32),
            pltpu.VMEM((_RED_WORDS,), jnp.float32),
            pltpu.SemaphoreType.DMA,
            pltpu.SemaphoreType.DMA,
        ],
    )(tgt_idx, ctx_idx, target_table, context_table)
    return out_flat.reshape(batch, NCTX)
